# speedup vs baseline: 1.1391x; 1.1391x over previous
import functools

import jax
import jax.numpy as jnp
from jax import lax
from jax.experimental import pallas as pl
from jax.experimental.pallas import tpu as pltpu
from jax.experimental.pallas import tpu_sc as plsc

HIDDEN = 768
BATCH = 4
SEQ = 2048
ROWS = BATCH * SEQ
EPS = 1e-12

_info = plsc.get_sparse_core_info()
_NC, _NS = _info.num_cores, _info.num_subcores
_NW = _NC * _NS
_RPW = ROWS // _NW
_CH = 64
_NCH = _RPW // _CH

_mesh = plsc.VectorSubcoreMesh(core_axis_name="c", subcore_axis_name="s")


@functools.partial(
    pl.kernel,
    mesh=_mesh,
    out_type=jax.ShapeDtypeStruct((ROWS, HIDDEN), jnp.float32),
    scratch_types=[
        pltpu.VMEM((_RPW,), jnp.int32),
        pltpu.VMEM((_CH, HIDDEN), jnp.float32),
        pltpu.VMEM((_CH, HIDDEN), jnp.float32),
        pltpu.SemaphoreType.DMA,
        pltpu.SemaphoreType.DMA,
    ],
)
def _gather_sc(table_hbm, ids_hbm, out_hbm, idx_v, buf0, buf1, sem0, sem1):
    wid = lax.axis_index("s") * _NC + lax.axis_index("c")
    base = wid * _RPW
    pltpu.sync_copy(ids_hbm.at[pl.ds(base, _RPW)], idx_v)
    bufs = (buf0, buf1)
    sems = (sem0, sem1)
    cur = pltpu.async_copy(table_hbm.at[idx_v.at[pl.ds(0, _CH)]], bufs[0], sems[0])
    for c in range(_NCH):
        nxt = None
        if c + 1 < _NCH:
            nxt = pltpu.async_copy(
                table_hbm.at[idx_v.at[pl.ds((c + 1) * _CH, _CH)]],
                bufs[(c + 1) % 2],
                sems[(c + 1) % 2],
            )
        cur.wait()
        pltpu.sync_copy(bufs[c % 2], out_hbm.at[pl.ds(base + c * _CH, _CH)])
        cur = nxt


def _ln_body(pos_ref, gamma_ref, beta_ref, emb_ref, out_ref):
    x = emb_ref[...] + pos_ref[...]
    mean = jnp.mean(x, axis=1, keepdims=True)
    xc = x - mean
    var = jnp.mean(xc * xc, axis=1, keepdims=True)
    inv = lax.rsqrt(var + EPS)
    out_ref[...] = xc * inv * gamma_ref[...] + beta_ref[...]


def kernel(input_ids, token_table, pos_table, gamma, beta):
    ids = input_ids.reshape(-1).astype(jnp.int32)
    emb = _gather_sc(token_table, ids)
    r = 256
    out = pl.pallas_call(
        _ln_body,
        grid=(ROWS // r,),
        in_specs=[
            pl.BlockSpec((r, HIDDEN), lambda i: (i % (SEQ // r), 0)),
            pl.BlockSpec((1, HIDDEN), lambda i: (0, 0)),
            pl.BlockSpec((1, HIDDEN), lambda i: (0, 0)),
            pl.BlockSpec((r, HIDDEN), lambda i: (i, 0)),
        ],
        out_specs=pl.BlockSpec((r, HIDDEN), lambda i: (i, 0)),
        out_shape=jax.ShapeDtypeStruct((ROWS, HIDDEN), jnp.float32),
    )(pos_table, gamma.reshape(1, HIDDEN), beta.reshape(1, HIDDEN), emb)
    return out.reshape(BATCH, SEQ, HIDDEN)


# speedup vs baseline: 1.1520x; 1.0113x over previous
import functools

import jax
import jax.numpy as jnp
from jax import lax
from jax.experimental import pallas as pl
from jax.experimental.pallas import tpu as pltpu
from jax.experimental.pallas import tpu_sc as plsc

HIDDEN = 768
BATCH = 4
SEQ = 2048
ROWS = BATCH * SEQ
EPS = 1e-12

_info = plsc.get_sparse_core_info()
_NC, _NS = _info.num_cores, _info.num_subcores
_NW = _NC * _NS
_RPW = ROWS // _NW
_CH = 64
_NCH = _RPW // _CH

_mesh = plsc.VectorSubcoreMesh(core_axis_name="c", subcore_axis_name="s")


@functools.partial(
    pl.kernel,
    mesh=_mesh,
    out_type=jax.ShapeDtypeStruct((ROWS, HIDDEN), jnp.float32),
    scratch_types=[
        pltpu.VMEM((_RPW,), jnp.int32),
        pltpu.VMEM((_CH, HIDDEN), jnp.float32),
        pltpu.VMEM((_CH, HIDDEN), jnp.float32),
        pltpu.SemaphoreType.DMA,
        pltpu.SemaphoreType.DMA,
    ],
)
def _gather_sc(table_hbm, ids_hbm, out_hbm, idx_v, buf0, buf1, sem0, sem1):
    wid = lax.axis_index("s") * _NC + lax.axis_index("c")
    base = wid * _RPW
    pltpu.sync_copy(ids_hbm.at[pl.ds(base, _RPW)], idx_v)
    bufs = (buf0, buf1)
    sems = (sem0, sem1)
    cur = pltpu.async_copy(table_hbm.at[idx_v.at[pl.ds(0, _CH)]], bufs[0], sems[0])
    for c in range(_NCH):
        nxt = None
        if c + 1 < _NCH:
            nxt = pltpu.async_copy(
                table_hbm.at[idx_v.at[pl.ds((c + 1) * _CH, _CH)]],
                bufs[(c + 1) % 2],
                sems[(c + 1) % 2],
            )
        cur.wait()
        pltpu.sync_copy(bufs[c % 2], out_hbm.at[pl.ds(base + c * _CH, _CH)])
        cur = nxt


def _ln_body(pos_ref, gamma_ref, beta_ref, emb_ref, out_ref):
    x = emb_ref[...] + pos_ref[...]
    mean = jnp.mean(x, axis=1, keepdims=True)
    xc = x - mean
    var = jnp.mean(xc * xc, axis=1, keepdims=True)
    inv = lax.rsqrt(var + EPS)
    out_ref[...] = xc * inv * gamma_ref[...] + beta_ref[...]


def kernel(input_ids, token_table, pos_table, gamma, beta):
    ids = input_ids.reshape(-1).astype(jnp.int32)
    emb = _gather_sc(token_table, ids)
    r = 256
    out = pl.pallas_call(
        _ln_body,
        grid=(SEQ // r, BATCH),
        in_specs=[
            pl.BlockSpec((r, HIDDEN), lambda j, b: (j, 0)),
            pl.BlockSpec((1, HIDDEN), lambda j, b: (0, 0)),
            pl.BlockSpec((1, HIDDEN), lambda j, b: (0, 0)),
            pl.BlockSpec((r, HIDDEN), lambda j, b: (b * (SEQ // r) + j, 0)),
        ],
        out_specs=pl.BlockSpec((r, HIDDEN), lambda j, b: (b * (SEQ // r) + j, 0)),
        out_shape=jax.ShapeDtypeStruct((ROWS, HIDDEN), jnp.float32),
    )(pos_table, gamma.reshape(1, HIDDEN), beta.reshape(1, HIDDEN), emb)
    return out.reshape(BATCH, SEQ, HIDDEN)


# speedup vs baseline: 1.4169x; 1.2300x over previous
import functools

import jax
import jax.numpy as jnp
from jax import lax
from jax.experimental import pallas as pl
from jax.experimental.pallas import tpu as pltpu
from jax.experimental.pallas import tpu_sc as plsc

HIDDEN = 768
BATCH = 4
SEQ = 2048
ROWS = BATCH * SEQ
EPS = 1e-12

_info = plsc.get_sparse_core_info()
_NC, _NS = _info.num_cores, _info.num_subcores
_NW = _NC * _NS
_RPW = ROWS // _NW
_CH = 64
_NCH = _RPW // _CH

_mesh = plsc.VectorSubcoreMesh(core_axis_name="c", subcore_axis_name="s")


@functools.partial(
    pl.kernel,
    mesh=_mesh,
    out_type=jax.ShapeDtypeStruct((ROWS, HIDDEN), jnp.float32),
    scratch_types=[
        pltpu.VMEM((_RPW,), jnp.int32),
        pltpu.VMEM((_CH, HIDDEN), jnp.float32),
        pltpu.VMEM((_CH, HIDDEN), jnp.float32),
        pltpu.SemaphoreType.DMA,
        pltpu.SemaphoreType.DMA,
    ],
)
def _gather_sc(table_hbm, ids_hbm, out_hbm, idx_v, buf0, buf1, sem0, sem1):
    wid = lax.axis_index("s") * _NC + lax.axis_index("c")
    base = wid * _RPW
    pltpu.sync_copy(ids_hbm.at[pl.ds(base, _RPW)], idx_v)
    bufs = (buf0, buf1)
    sems = (sem0, sem1)
    cur = pltpu.async_copy(table_hbm.at[idx_v.at[pl.ds(0, _CH)]], bufs[0], sems[0])
    for c in range(_NCH):
        nxt = None
        if c + 1 < _NCH:
            nxt = pltpu.async_copy(
                table_hbm.at[idx_v.at[pl.ds((c + 1) * _CH, _CH)]],
                bufs[(c + 1) % 2],
                sems[(c + 1) % 2],
            )
        cur.wait()
        pltpu.sync_copy(bufs[c % 2], out_hbm.at[pl.ds(base + c * _CH, _CH)])
        cur = nxt


def _ln_body(pos_ref, gamma_ref, beta_ref, emb_ref, out_ref):
    x = emb_ref[...] + pos_ref[...]
    mean = jnp.mean(x, axis=1, keepdims=True)
    xc = x - mean
    var = jnp.mean(xc * xc, axis=1, keepdims=True)
    inv = lax.rsqrt(var + EPS)
    out_ref[...] = xc * inv * gamma_ref[...] + beta_ref[...]


def kernel(input_ids, token_table, pos_table, gamma, beta):
    ids = input_ids.reshape(-1).astype(jnp.int32)
    emb = _gather_sc(token_table, ids)
    r = 1024
    out = pl.pallas_call(
        _ln_body,
        grid=(SEQ // r, BATCH),
        in_specs=[
            pl.BlockSpec((r, HIDDEN), lambda j, b: (j, 0)),
            pl.BlockSpec((1, HIDDEN), lambda j, b: (0, 0)),
            pl.BlockSpec((1, HIDDEN), lambda j, b: (0, 0)),
            pl.BlockSpec((r, HIDDEN), lambda j, b: (b * (SEQ // r) + j, 0)),
        ],
        out_specs=pl.BlockSpec((r, HIDDEN), lambda j, b: (b * (SEQ // r) + j, 0)),
        out_shape=jax.ShapeDtypeStruct((ROWS, HIDDEN), jnp.float32),
    )(pos_table, gamma.reshape(1, HIDDEN), beta.reshape(1, HIDDEN), emb)
    return out.reshape(BATCH, SEQ, HIDDEN)


# speedup vs baseline: 1.4352x; 1.0130x over previous
import functools

import jax
import jax.numpy as jnp
from jax import lax
from jax.experimental import pallas as pl
from jax.experimental.pallas import tpu as pltpu
from jax.experimental.pallas import tpu_sc as plsc

HIDDEN = 768
BATCH = 4
SEQ = 2048
ROWS = BATCH * SEQ
EPS = 1e-12

_info = plsc.get_sparse_core_info()
_NC, _NS = _info.num_cores, _info.num_subcores
_NW = _NC * _NS
_RPW = ROWS // _NW
_CH = 64
_NCH = _RPW // _CH

_mesh = plsc.VectorSubcoreMesh(core_axis_name="c", subcore_axis_name="s")


@functools.partial(
    pl.kernel,
    mesh=_mesh,
    out_type=jax.ShapeDtypeStruct((ROWS, HIDDEN), jnp.float32),
    scratch_types=[
        pltpu.VMEM((_RPW,), jnp.int32),
        pltpu.VMEM((_CH, HIDDEN), jnp.float32),
        pltpu.VMEM((_CH, HIDDEN), jnp.float32),
        pltpu.SemaphoreType.DMA,
        pltpu.SemaphoreType.DMA,
    ],
)
def _gather_sc(table_hbm, ids_hbm, out_hbm, idx_v, buf0, buf1, sem0, sem1):
    wid = lax.axis_index("s") * _NC + lax.axis_index("c")
    base = wid * _RPW
    pltpu.sync_copy(ids_hbm.at[pl.ds(base, _RPW)], idx_v)
    bufs = (buf0, buf1)
    sems = (sem0, sem1)
    cur = pltpu.async_copy(table_hbm.at[idx_v.at[pl.ds(0, _CH)]], bufs[0], sems[0])
    for c in range(_NCH):
        nxt = None
        if c + 1 < _NCH:
            nxt = pltpu.async_copy(
                table_hbm.at[idx_v.at[pl.ds((c + 1) * _CH, _CH)]],
                bufs[(c + 1) % 2],
                sems[(c + 1) % 2],
            )
        cur.wait()
        pltpu.sync_copy(bufs[c % 2], out_hbm.at[pl.ds(base + c * _CH, _CH)])
        cur = nxt


def _ln_body(pos_ref, gamma_ref, beta_ref, emb_ref, out_ref):
    x = emb_ref[...] + pos_ref[...]
    mean = jnp.mean(x, axis=1, keepdims=True)
    xc = x - mean
    var = jnp.mean(xc * xc, axis=1, keepdims=True)
    inv = lax.rsqrt(var + EPS)
    out_ref[...] = xc * inv * gamma_ref[...] + beta_ref[...]


def kernel(input_ids, token_table, pos_table, gamma, beta):
    ids = input_ids.reshape(-1).astype(jnp.int32)
    emb = _gather_sc(token_table, ids)
    r = 2048
    out = pl.pallas_call(
        _ln_body,
        grid=(SEQ // r, BATCH),
        in_specs=[
            pl.BlockSpec((r, HIDDEN), lambda j, b: (j, 0)),
            pl.BlockSpec((1, HIDDEN), lambda j, b: (0, 0)),
            pl.BlockSpec((1, HIDDEN), lambda j, b: (0, 0)),
            pl.BlockSpec((r, HIDDEN), lambda j, b: (b * (SEQ // r) + j, 0)),
        ],
        out_specs=pl.BlockSpec((r, HIDDEN), lambda j, b: (b * (SEQ // r) + j, 0)),
        out_shape=jax.ShapeDtypeStruct((ROWS, HIDDEN), jnp.float32),
    )(pos_table, gamma.reshape(1, HIDDEN), beta.reshape(1, HIDDEN), emb)
    return out.reshape(BATCH, SEQ, HIDDEN)
